# baseline (speedup 1.0000x reference)
import jax
import jax.numpy as jnp
from jax.experimental import pallas as pl
from jax.experimental.pallas import tpu as pltpu

_EMBED = 768
_NE = 64
_CH = 512
_NCH = 8
_BLK = _CH * _NCH
_SLOTS = 2 * _NCH


def _chunk_copy(x_hbm, buf, sem, step, slot_base, j):
    return pltpu.make_async_copy(
        x_hbm.at[pl.ds(step * _BLK + j * _CH, _CH), :],
        buf.at[slot_base + j],
        sem.at[slot_base + j],
    )


def _router_body(x_hbm, wt_ref, w_out_ref, i_out_ref, buf, sem):
    i = pl.program_id(0)
    nsteps = pl.num_programs(0)

    @pl.when(i == 0)
    def _():
        for j in range(_NCH):
            _chunk_copy(x_hbm, buf, sem, i, 0, j).start()

    @pl.when(i < nsteps - 1)
    def _():
        nxt_base = ((i + 1) % 2) * _NCH
        for j in range(_NCH):
            _chunk_copy(x_hbm, buf, sem, i + 1, nxt_base, j).start()

    cur_base = (i % 2) * _NCH
    wt = wt_ref[...].astype(jnp.bfloat16)
    for j in range(_NCH):
        _chunk_copy(x_hbm, buf, sem, i, cur_base, j).wait()
        x = buf[cur_base + j].astype(jnp.bfloat16)
        logits = jax.lax.dot_general(
            x, wt, (((1,), (0,)), ((), ())),
            preferred_element_type=jnp.float32)
        m1 = jnp.max(logits, axis=1, keepdims=True)
        i1 = jnp.argmax(logits, axis=1).astype(jnp.int32)[:, None]
        masked = jnp.where(logits == m1, -jnp.inf, logits)
        m2 = jnp.max(masked, axis=1, keepdims=True)
        i2 = jnp.argmax(masked, axis=1).astype(jnp.int32)[:, None]
        e = jnp.exp(m2 - m1)
        w1 = 1.0 / (1.0 + e)
        sl = pl.ds(j * _CH, _CH)
        w_out_ref[sl, 0:1] = w1
        w_out_ref[sl, 1:2] = e * w1
        i_out_ref[sl, 0:1] = i1
        i_out_ref[sl, 1:2] = i2


def kernel(x, W_gate):
    B, L, D = x.shape
    T = B * L
    xt = x.reshape(T, D)
    wt = W_gate.T
    w_out, i_out = pl.pallas_call(
        _router_body,
        grid=(T // _BLK,),
        in_specs=[
            pl.BlockSpec(memory_space=pl.ANY),
            pl.BlockSpec((D, _NE), lambda i: (0, 0)),
        ],
        out_specs=[
            pl.BlockSpec((_BLK, 2), lambda i: (i, 0)),
            pl.BlockSpec((_BLK, 2), lambda i: (i, 0)),
        ],
        out_shape=[
            jax.ShapeDtypeStruct((T, 2), jnp.float32),
            jax.ShapeDtypeStruct((T, 2), jnp.int32),
        ],
        scratch_shapes=[
            pltpu.VMEM((_SLOTS, _CH, _EMBED), jnp.float32),
            pltpu.SemaphoreType.DMA((_SLOTS,)),
        ],
    )(xt, wt)
    return (w_out.reshape(B, L, 2), i_out.reshape(B, L, 2))


# speedup vs baseline: 1.1774x; 1.1774x over previous
import jax
import jax.numpy as jnp
from jax.experimental import pallas as pl

_EMBED = 768
_NE = 64
_BLK = 4096
_NSPLIT = 8
_SUB = _BLK // _NSPLIT


def _router_body(*refs):
    x_refs = refs[:_NSPLIT]
    wt_ref = refs[_NSPLIT]
    w_out_ref, i_out_ref = refs[_NSPLIT + 1:]
    wt = wt_ref[...].astype(jnp.bfloat16)
    for j in range(_NSPLIT):
        x = x_refs[j][...].astype(jnp.bfloat16)
        logits = jax.lax.dot_general(
            x, wt, (((1,), (0,)), ((), ())),
            preferred_element_type=jnp.float32)
        m1 = jnp.max(logits, axis=1, keepdims=True)
        i1 = jnp.argmax(logits, axis=1).astype(jnp.int32)[:, None]
        masked = jnp.where(logits == m1, -jnp.inf, logits)
        m2 = jnp.max(masked, axis=1, keepdims=True)
        i2 = jnp.argmax(masked, axis=1).astype(jnp.int32)[:, None]
        e = jnp.exp(m2 - m1)
        w1 = 1.0 / (1.0 + e)
        sl = pl.ds(j * _SUB, _SUB)
        w_out_ref[sl, 0:1] = w1
        w_out_ref[sl, 1:2] = e * w1
        i_out_ref[sl, 0:1] = i1
        i_out_ref[sl, 1:2] = i2


def kernel(x, W_gate):
    B, L, D = x.shape
    T = B * L
    xt = x.reshape(T, D)
    wt = W_gate.T
    in_specs = [
        pl.BlockSpec((_SUB, D), lambda i, j=j: (i * _NSPLIT + j, 0))
        for j in range(_NSPLIT)
    ]
    in_specs.append(pl.BlockSpec((D, _NE), lambda i: (0, 0)))
    w_out, i_out = pl.pallas_call(
        _router_body,
        grid=(T // _BLK,),
        in_specs=in_specs,
        out_specs=[
            pl.BlockSpec((_BLK, 2), lambda i: (i, 0)),
            pl.BlockSpec((_BLK, 2), lambda i: (i, 0)),
        ],
        out_shape=[
            jax.ShapeDtypeStruct((T, 2), jnp.float32),
            jax.ShapeDtypeStruct((T, 2), jnp.int32),
        ],
    )(*([xt] * _NSPLIT), wt)
    return (w_out.reshape(B, L, 2), i_out.reshape(B, L, 2))
